# baseline (speedup 1.0000x reference)
import jax
import jax.numpy as jnp
from jax.experimental import pallas as pl
from jax.experimental.pallas import tpu as pltpu

_M_TILE = 1024
_K_TILE = 512


def _expert_mlp_kernel(x_ref, w_ref, b_ref, o_ref):
    k = pl.program_id(1)
    xb = x_ref[...].astype(jnp.bfloat16)
    wb = w_ref[0].astype(jnp.bfloat16)
    acc = jnp.dot(xb, wb, preferred_element_type=jnp.float32)

    @pl.when(k == 0)
    def _init():
        o_ref[...] = acc + b_ref[0]

    @pl.when(k != 0)
    def _accum():
        o_ref[...] += acc


def kernel(x, w_gate, expert_w, expert_b):
    b, s, d_in = x.shape
    e, _, d_out = expert_w.shape
    m = b * s
    xm = x.reshape(m, d_in)
    eb3 = expert_b.reshape(e, 1, d_out)
    out = pl.pallas_call(
        _expert_mlp_kernel,
        grid=(m // _M_TILE, d_in // _K_TILE),
        in_specs=[
            pl.BlockSpec((_M_TILE, _K_TILE), lambda i, k: (i, k)),
            pl.BlockSpec((1, _K_TILE, d_out), lambda i, k: (1, k, 0)),
            pl.BlockSpec((1, 1, d_out), lambda i, k: (1, 0, 0)),
        ],
        out_specs=pl.BlockSpec((_M_TILE, d_out), lambda i, k: (i, 0)),
        out_shape=jax.ShapeDtypeStruct((m, d_out), x.dtype),
        compiler_params=pltpu.CompilerParams(
            dimension_semantics=("parallel", "arbitrary"),
        ),
    )(xm, expert_w, eb3)
    return out.reshape(b, s, d_out)


# speedup vs baseline: 1.3873x; 1.3873x over previous
import jax
import jax.numpy as jnp
from jax.experimental import pallas as pl
from jax.experimental.pallas import tpu as pltpu

_M_TILE = 1024


def _expert_mlp_kernel(x_ref, w_ref, b_ref, o_ref):
    xb = x_ref[...].astype(jnp.bfloat16)
    wb = w_ref[0].astype(jnp.bfloat16)
    o_ref[...] = (
        jnp.dot(xb, wb, preferred_element_type=jnp.float32)
        + b_ref[0]
    )


def kernel(x, w_gate, expert_w, expert_b):
    b, s, d_in = x.shape
    e, _, d_out = expert_w.shape
    m = b * s
    xm = x.reshape(m, d_in)
    eb3 = expert_b.reshape(e, 1, d_out)
    out = pl.pallas_call(
        _expert_mlp_kernel,
        grid=(m // _M_TILE,),
        in_specs=[
            pl.BlockSpec((_M_TILE, d_in), lambda i: (i, 0)),
            pl.BlockSpec((1, d_in, d_out), lambda i: (1, 0, 0)),
            pl.BlockSpec((1, 1, d_out), lambda i: (1, 0, 0)),
        ],
        out_specs=pl.BlockSpec((_M_TILE, d_out), lambda i: (i, 0)),
        out_shape=jax.ShapeDtypeStruct((m, d_out), x.dtype),
        compiler_params=pltpu.CompilerParams(
            dimension_semantics=("parallel",),
        ),
    )(xm, expert_w, eb3)
    return out.reshape(b, s, d_out)
